# baseline (speedup 1.0000x reference)
import jax
import jax.numpy as jnp
from jax import lax
from jax.experimental import pallas as pl
from jax.experimental.pallas import tpu as pltpu
from jax.experimental.pallas import tpu_sc as plsc

T = 4
B = 128
VOCAB = 100000
ATTN = 200
OOV = 100
VEXT = VOCAB + OOV
ROWS = T * B
LANES = 16
NW = 32
ROWS_PER_W = ROWS // NW
BUF = 100112

IW = 112
SPLIT = 104

VB = 2048
NVB = (VEXT + VB - 1) // VB


def _tc_body(pg_ref, vd_ref, out_ref):
    j = pl.program_id(0)
    v = j * VB + lax.broadcasted_iota(jnp.int32, (VB, T, 128), 0)
    pg = pg_ref[...][None]
    vals = pg * jnp.transpose(vd_ref[...], (1, 0, 2))
    out_ref[...] = jnp.where(v < VOCAB, vals, 0.0)


def _sc_body(attn_hbm, pg_hbm, idx_hbm, out_ref,
             ivb, avb, ovb, buf, pgv, sem):
    wid = lax.axis_index("s") * 2 + lax.axis_index("c")
    lanes = lax.iota(jnp.int32, LANES)
    zf = jnp.zeros((LANES,), jnp.float32)
    zi = jnp.zeros((LANES,), jnp.int32)
    row0 = wid * ROWS_PER_W
    b0 = lax.rem(row0, B)
    RW = 2 * IW

    pltpu.sync_copy(pg_hbm, pgv.at[pl.ds(0, ROWS)])

    def stage(r, c):
        pltpu.async_copy(idx_hbm.at[pl.ds((b0 + r) * ATTN, SPLIT)],
                         ivb.at[pl.ds(r * RW, SPLIT)], sem)
        pltpu.async_copy(idx_hbm.at[pl.ds((b0 + r) * ATTN + SPLIT,
                                          ATTN - SPLIT)],
                         ivb.at[pl.ds(r * RW + IW, ATTN - SPLIT)], sem)
        pltpu.async_copy(attn_hbm.at[pl.ds((row0 + r) * ATTN, SPLIT)],
                         avb.at[pl.ds(r * RW, SPLIT)], sem)
        pltpu.async_copy(attn_hbm.at[pl.ds((row0 + r) * ATTN + SPLIT,
                                           ATTN - SPLIT)],
                         avb.at[pl.ds(r * RW + IW, ATTN - SPLIT)], sem)
        return c
    lax.fori_loop(0, ROWS_PER_W, stage, 0)

    def stage_drain(r, c):
        pltpu.make_async_copy(idx_hbm.at[pl.ds((b0 + r) * ATTN, SPLIT)],
                              ivb.at[pl.ds(r * RW, SPLIT)], sem).wait()
        pltpu.make_async_copy(idx_hbm.at[pl.ds((b0 + r) * ATTN + SPLIT,
                                               ATTN - SPLIT)],
                              ivb.at[pl.ds(r * RW + IW, ATTN - SPLIT)],
                              sem).wait()
        pltpu.make_async_copy(attn_hbm.at[pl.ds((row0 + r) * ATTN, SPLIT)],
                              avb.at[pl.ds(r * RW, SPLIT)], sem).wait()
        pltpu.make_async_copy(attn_hbm.at[pl.ds((row0 + r) * ATTN + SPLIT,
                                                ATTN - SPLIT)],
                              avb.at[pl.ds(r * RW + IW, ATTN - SPLIT)],
                              sem).wait()
        return c
    lax.fori_loop(0, ROWS_PER_W, stage_drain, 0)

    def convert(r, c):
        row = row0 + r
        o0 = r * RW
        ivb[pl.ds(o0 + 96, LANES)] = jnp.where(
            lanes < SPLIT - 96, ivb[pl.ds(o0 + 96, LANES)], zi)
        ivb[pl.ds(o0 + IW + 96, LANES)] = zi
        avb[pl.ds(o0 + 96, LANES)] = jnp.where(
            lanes < SPLIT - 96, avb[pl.ds(o0 + 96, LANES)], zf)
        avb[pl.ds(o0 + IW + 96, LANES)] = zf
        for cc in range(RW // LANES):
            o = o0 + cc * LANES
            ivb[pl.ds(o, LANES)] = ivb[pl.ds(o, LANES)] * 512 + row
        return c
    lax.fori_loop(0, ROWS_PER_W, convert, 0)

    pltpu.async_copy(out_ref.at[ivb], ovb, sem).wait()

    def process(r, c):
        row = row0 + r
        o0 = r * RW
        pgwin = pgv[pl.ds(row, LANES)]
        omg = jnp.ones((LANES,), jnp.float32) - (zf + pgwin[0])
        for cc in range(RW // LANES):
            o = o0 + cc * LANES
            vloc = lax.shift_right_logical(ivb[pl.ds(o, LANES)], 9)
            plsc.store_scatter(buf, [vloc], ovb[pl.ds(o, LANES)])
        for cc in range(RW // LANES):
            o = o0 + cc * LANES
            vloc = lax.shift_right_logical(ivb[pl.ds(o, LANES)], 9)
            vals = avb[pl.ds(o, LANES)] * omg
            for lane in range(LANES):
                plsc.addupdate_scatter(buf, [vloc], vals,
                                       mask=lanes == lane)
        for cc in range(RW // LANES):
            o = o0 + cc * LANES
            vloc = lax.shift_right_logical(ivb[pl.ds(o, LANES)], 9)
            ovb[pl.ds(o, LANES)] = plsc.load_gather(buf, [vloc])
        return c
    lax.fori_loop(0, ROWS_PER_W, process, 0)

    pltpu.async_copy(ovb, out_ref.at[ivb], sem).wait()


@jax.jit
def _final_dist(vocab_dists, attn_dists, p_gens, enc_batch_extend_vocab):
    vocab_t = jnp.transpose(vocab_dists, (0, 2, 1))
    pg2 = p_gens.reshape(T, B)

    dense = pl.pallas_call(
        _tc_body,
        grid=(NVB,),
        in_specs=[
            pl.BlockSpec((T, B), lambda j: (0, 0)),
            pl.BlockSpec((T, VB, B), lambda j: (0, j, 0)),
        ],
        out_specs=pl.BlockSpec((VB, T, B), lambda j: (j, 0, 0)),
        out_shape=jax.ShapeDtypeStruct((VEXT, T, B), jnp.float32),
        compiler_params=pltpu.CompilerParams(
            dimension_semantics=("arbitrary",)),
    )(pg2, vocab_t)

    attn1 = attn_dists.reshape(ROWS * ATTN)
    pg1 = p_gens.reshape(ROWS)
    idx1 = enc_batch_extend_vocab.reshape(B * ATTN)

    mesh = plsc.VectorSubcoreMesh(core_axis_name="c", subcore_axis_name="s")
    rmw = pl.kernel(
        _sc_body,
        out_type=(),
        mesh=mesh,
        compiler_params=pltpu.CompilerParams(needs_layout_passes=False),
        scratch_types=[
            pltpu.VMEM((ROWS_PER_W * 2 * IW,), jnp.int32),
            pltpu.VMEM((ROWS_PER_W * 2 * IW,), jnp.float32),
            pltpu.VMEM((ROWS_PER_W * 2 * IW,), jnp.float32),
            pltpu.VMEM((BUF,), jnp.float32),
            pltpu.VMEM((ROWS + LANES,), jnp.float32),
            pltpu.SemaphoreType.DMA,
        ],
    )
    ref = jax.new_ref(dense.reshape(VEXT * ROWS))
    rmw(attn1, pg1, idx1, ref)
    out_t = ref[...].reshape(VEXT, T, B)
    return jnp.transpose(out_t, (1, 2, 0))


def kernel(vocab_dists, attn_dists, p_gens, enc_batch_extend_vocab):
    return _final_dist(vocab_dists, attn_dists, p_gens,
                       enc_batch_extend_vocab)


# speedup vs baseline: 1.0626x; 1.0626x over previous
import jax
import jax.numpy as jnp
from jax import lax
from jax.experimental import pallas as pl
from jax.experimental.pallas import tpu as pltpu
from jax.experimental.pallas import tpu_sc as plsc

T = 4
B = 128
VOCAB = 100000
ATTN = 200
OOV = 100
VEXT = VOCAB + OOV
ROWS = T * B
LANES = 16
NW = 32
NBANDS = ROWS // 8
BPW = NBANDS // NW

VTILE_FULL = VOCAB // 128
VTILE_REM = VOCAB % 128
OTILES = (VEXT + 127) // 128
OTILE_REM = VEXT % 128

NT = 112
NSEG = 6
NT_LAST = OTILES - NSEG * NT
APAD = 208


def _sc_body(vocab_hbm, attn_hbm, pg_hbm, idx_hbm, out_hbm,
             buf, iv2, av2, pgv, sem):
    wid = lax.axis_index("s") * 2 + lax.axis_index("c")
    lanes = lax.iota(jnp.int32, LANES)
    zf = jnp.zeros((LANES,), jnp.float32)
    zi = jnp.zeros((LANES,), jnp.int32)

    pltpu.sync_copy(pg_hbm, pgv.at[pl.ds(0, ROWS)])
    for r in range(8):
        iv2[pl.ds(r * APAD + 192, LANES)] = zi
        av2[pl.ds(r * APAD + 192, LANES)] = zf

    def in_tile(g, t, tg):
        tt = lax.div(g, 16)
        bb = lax.rem(g, 16) * 8
        return (vocab_hbm.at[tt, pl.ds(bb, 8), pl.ds(tg * 128, 128)],
                buf.at[t])

    def out_tile(g, t, tg):
        return (buf.at[t],
                out_hbm.at[pl.ds(g * 8, 8), pl.ds(tg * 128, 128)])

    def mul_seg(g, nt):
        def body(t, c):
            for r in range(8):
                pgwin = pgv[pl.ds(g * 8 + r, LANES)]
                pgvec = zf + pgwin[0]
                for j in range(8):
                    buf[t, r, pl.ds(j * LANES, LANES)] = (
                        buf[t, r, pl.ds(j * LANES, LANES)] * pgvec)
            return c
        lax.fori_loop(0, nt, body, 0)

    def scatter_seg(g, t0, nt):
        def rbody(r, c):
            pgwin = pgv[pl.ds(g * 8 + r, LANES)]
            omg = jnp.ones((LANES,), jnp.float32) - (zf + pgwin[0])
            r16 = zi + r

            def cbody(cc, c2):
                ivc = iv2[pl.ds(r * APAD + cc * LANES, LANES)]
                vals = av2[pl.ds(r * APAD + cc * LANES, LANES)] * omg
                tloc = lax.shift_right_logical(ivc, 7) - t0
                cl = lax.bitwise_and(ivc, 127)
                valid = (tloc >= 0) & (tloc < nt)
                for lane in range(LANES):
                    plsc.addupdate_scatter(
                        buf, [tloc, r16, cl], vals,
                        mask=valid & (lanes == lane))
                return c2
            lax.fori_loop(0, APAD // LANES, cbody, 0)
            return c
        lax.fori_loop(0, 8, rbody, 0)

    for i in range(BPW):
        g = wid * BPW + i
        row0 = g * 8
        b0 = lax.rem(row0, B)

        for r in range(8):
            pltpu.sync_copy(idx_hbm.at[pl.ds((b0 + r) * ATTN, 104)],
                            iv2.at[pl.ds(r * APAD, 104)])
            pltpu.sync_copy(idx_hbm.at[pl.ds((b0 + r) * ATTN + 104, 96)],
                            iv2.at[pl.ds(r * APAD + 104, 96)])
            pltpu.sync_copy(attn_hbm.at[pl.ds((row0 + r) * ATTN, 104)],
                            av2.at[pl.ds(r * APAD, 104)])
            pltpu.sync_copy(attn_hbm.at[pl.ds((row0 + r) * ATTN + 104, 96)],
                            av2.at[pl.ds(r * APAD + 104, 96)])

        def seg_body(s, c):
            t0 = s * NT

            def fire(t, c2):
                pltpu.async_copy(*in_tile(g, t, t0 + t), sem)
                return c2
            lax.fori_loop(0, NT, fire, 0)

            def drain_mul(t, c2):
                pltpu.make_async_copy(*in_tile(g, t, t0 + t), sem).wait()
                for r in range(8):
                    pgwin = pgv[pl.ds(g * 8 + r, LANES)]
                    pgvec = jnp.zeros((LANES,), jnp.float32) + pgwin[0]
                    for j in range(8):
                        buf[t, r, pl.ds(j * LANES, LANES)] = (
                            buf[t, r, pl.ds(j * LANES, LANES)] * pgvec)
                return c2
            lax.fori_loop(0, NT, drain_mul, 0)

            scatter_seg(g, t0, NT)

            def ofire(t, c2):
                pltpu.async_copy(*out_tile(g, t, t0 + t), sem)
                return c2
            lax.fori_loop(0, NT, ofire, 0)

            def odrain(t, c2):
                pltpu.make_async_copy(*out_tile(g, t, t0 + t), sem).wait()
                return c2
            lax.fori_loop(0, NT, odrain, 0)
            return c
        lax.fori_loop(0, NSEG, seg_body, 0)

        t0 = NSEG * NT
        nfull = VTILE_FULL - t0

        def lfire(t, c2):
            pltpu.async_copy(*in_tile(g, t, t0 + t), sem)
            return c2
        lax.fori_loop(0, nfull, lfire, 0)
        tt = lax.div(g, 16)
        bb = lax.rem(g, 16) * 8
        for r in range(8):
            pltpu.async_copy(
                vocab_hbm.at[tt, bb + r, pl.ds(VTILE_FULL * 128, VTILE_REM)],
                buf.at[nfull, r, pl.ds(0, VTILE_REM)], sem)

        def ldrain(t, c2):
            pltpu.make_async_copy(*in_tile(g, t, t0 + t), sem).wait()
            return c2
        lax.fori_loop(0, nfull, ldrain, 0)
        for r in range(8):
            pltpu.make_async_copy(
                vocab_hbm.at[tt, bb + r, pl.ds(VTILE_FULL * 128, VTILE_REM)],
                buf.at[nfull, r, pl.ds(0, VTILE_REM)], sem).wait()

        for r in range(8):
            for j in range(VTILE_REM // LANES, 8):
                buf[nfull, r, pl.ds(j * LANES, LANES)] = zf
            for j in range(8):
                buf[nfull + 1, r, pl.ds(j * LANES, LANES)] = zf

        mul_seg(g, NT_LAST)
        scatter_seg(g, t0, NT_LAST)

        def lofire(t, c2):
            pltpu.async_copy(*out_tile(g, t, t0 + t), sem)
            return c2
        lax.fori_loop(0, nfull + 1, lofire, 0)
        for r in range(8):
            pltpu.async_copy(
                buf.at[nfull + 1, r, pl.ds(0, OTILE_REM)],
                out_hbm.at[row0 + r, pl.ds((OTILES - 1) * 128, OTILE_REM)],
                sem)

        def lodrain(t, c2):
            pltpu.make_async_copy(*out_tile(g, t, t0 + t), sem).wait()
            return c2
        lax.fori_loop(0, nfull + 1, lodrain, 0)
        for r in range(8):
            pltpu.make_async_copy(
                buf.at[nfull + 1, r, pl.ds(0, OTILE_REM)],
                out_hbm.at[row0 + r, pl.ds((OTILES - 1) * 128, OTILE_REM)],
                sem).wait()


@jax.jit
def _final_dist(vocab_dists, attn_dists, p_gens, enc_batch_extend_vocab):
    attn1 = attn_dists.reshape(ROWS * ATTN)
    pg1 = p_gens.reshape(ROWS)
    idx1 = enc_batch_extend_vocab.reshape(B * ATTN)

    mesh = plsc.VectorSubcoreMesh(core_axis_name="c", subcore_axis_name="s")
    run = pl.kernel(
        _sc_body,
        out_type=jax.ShapeDtypeStruct((ROWS, VEXT), jnp.float32),
        mesh=mesh,
        compiler_params=pltpu.CompilerParams(needs_layout_passes=False),
        scratch_types=[
            pltpu.VMEM((NT, 8, 128), jnp.float32),
            pltpu.VMEM((8 * APAD,), jnp.int32),
            pltpu.VMEM((8 * APAD,), jnp.float32),
            pltpu.VMEM((ROWS + LANES,), jnp.float32),
            pltpu.SemaphoreType.DMA,
        ],
    )
    out2 = run(vocab_dists, attn1, pg1, idx1)
    return out2.reshape(T, B, VEXT)


def kernel(vocab_dists, attn_dists, p_gens, enc_batch_extend_vocab):
    return _final_dist(vocab_dists, attn_dists, p_gens,
                       enc_batch_extend_vocab)
